# baseline (speedup 1.0000x reference)
import functools

import jax
import jax.numpy as jnp
from jax import lax
from jax.experimental import pallas as pl
from jax.experimental.pallas import tpu as pltpu
from jax.experimental.pallas import tpu_sc as plsc

_NC = 2
_NS = 16
_LANES = 16
_NW = _NC * _NS
_CH = 128
_CHE = 64


def _ceil_to(x, m):
    return -(-x // m) * m


def _embed_body(ent_hbm, act_hbm, eemb_hbm, aemb_hbm, xe_hbm, xa_hbm,
                ie0, ia0, ie1, ia1, re0, ra0, re1, ra1,
                se0, sa0, se1, sa1, swe, swa):
    npad = xe_hbm.shape[0]
    per_w = npad // _NW
    w = lax.axis_index("s") * _NC + lax.axis_index("c")
    base_w = w * per_w
    n_it = per_w // _CHE
    idxs = ((ie0, ia0), (ie1, ia1))
    rows = ((re0, ra0), (re1, ra1))
    sems = ((se0, sa0), (se1, sa1))

    def _load(k, sl):
        base = base_w + k * _CHE
        pltpu.sync_copy(ent_hbm.at[pl.ds(base, _CHE)], idxs[sl][0])
        pltpu.sync_copy(act_hbm.at[pl.ds(base, _CHE)], idxs[sl][1])

    def _fire(sl):
        pltpu.async_copy(eemb_hbm.at[idxs[sl][0]], rows[sl][0], sems[sl][0])
        pltpu.async_copy(aemb_hbm.at[idxs[sl][1]], rows[sl][1], sems[sl][1])

    _load(0, 0)
    _fire(0)
    if n_it > 1:
        _load(1, 1)
        _fire(1)
    for k in range(n_it):
        sl = k % 2
        base = base_w + k * _CHE
        pltpu.make_async_copy(eemb_hbm.at[idxs[sl][0]], rows[sl][0],
                              sems[sl][0]).wait()
        pltpu.make_async_copy(aemb_hbm.at[idxs[sl][1]], rows[sl][1],
                              sems[sl][1]).wait()
        pltpu.async_copy(rows[sl][0], xe_hbm.at[pl.ds(base, _CHE)], swe)
        pltpu.async_copy(rows[sl][1], xa_hbm.at[pl.ds(base, _CHE)], swa)
        if k + 2 < n_it:
            _load(k + 2, sl)
            pltpu.make_async_copy(rows[sl][0], xe_hbm.at[pl.ds(base, _CHE)],
                                  swe).wait()
            pltpu.make_async_copy(rows[sl][1], xa_hbm.at[pl.ds(base, _CHE)],
                                  swa).wait()
            _fire(sl)
    for k in (n_it - 2, n_it - 1):
        if k < 0:
            continue
        sl = k % 2
        base = base_w + k * _CHE
        pltpu.make_async_copy(rows[sl][0], xe_hbm.at[pl.ds(base, _CHE)],
                              swe).wait()
        pltpu.make_async_copy(rows[sl][1], xa_hbm.at[pl.ds(base, _CHE)],
                              swa).wait()


def _embed_gather(ent_p, act_p, entity_emb, action_emb, npad):
    mesh = plsc.VectorSubcoreMesh(core_axis_name="c", subcore_axis_name="s")
    f = pl.kernel(
        _embed_body,
        out_type=(jax.ShapeDtypeStruct((npad, 128), jnp.float32),
                  jax.ShapeDtypeStruct((npad, 128), jnp.float32)),
        mesh=mesh,
        scratch_types=(
            pltpu.VMEM((_CHE,), jnp.int32),
            pltpu.VMEM((_CHE,), jnp.int32),
            pltpu.VMEM((_CHE,), jnp.int32),
            pltpu.VMEM((_CHE,), jnp.int32),
            pltpu.VMEM((_CHE, 128), jnp.float32),
            pltpu.VMEM((_CHE, 128), jnp.float32),
            pltpu.VMEM((_CHE, 128), jnp.float32),
            pltpu.VMEM((_CHE, 128), jnp.float32),
            pltpu.SemaphoreType.DMA,
            pltpu.SemaphoreType.DMA,
            pltpu.SemaphoreType.DMA,
            pltpu.SemaphoreType.DMA,
            pltpu.SemaphoreType.DMA,
            pltpu.SemaphoreType.DMA,
        ),
    )
    return f(ent_p, act_p, entity_emb, action_emb)


def _proj_kernel(xa_ref, xb_ref, sec_ref, bias_ref, w_ref, asrc_ref, adst_ref,
                 wsec_ref, h0_ref, h1_ref, est_ref, edt_ref, *, apply_elu):
    x = jnp.concatenate([xa_ref[...], xb_ref[...]], axis=1)
    if apply_elu:
        t = x + bias_ref[...]
        x = jnp.where(t > 0, t, jnp.exp(jnp.minimum(t, 0.0)) - 1.0)
    h = jnp.dot(x, w_ref[...], preferred_element_type=jnp.float32)
    h0_ref[...] = h[:, :128]
    h1_ref[...] = h[:, 128:]
    est_ref[...] = jnp.dot(h, asrc_ref[...], preferred_element_type=jnp.float32)
    edt_ref[...] = (jnp.dot(h, adst_ref[...], preferred_element_type=jnp.float32)
                    + jnp.dot(sec_ref[...], wsec_ref[...],
                              preferred_element_type=jnp.float32))


def _proj(xa, xb, sec_p, bias, W, A_src, A_dst, Wsec_p, apply_elu):
    npad = xa.shape[0]
    br = 512
    grid = (npad // br,)
    f = pl.pallas_call(
        functools.partial(_proj_kernel, apply_elu=apply_elu),
        grid=grid,
        in_specs=[
            pl.BlockSpec((br, 128), lambda i: (i, 0)),
            pl.BlockSpec((br, 128), lambda i: (i, 0)),
            pl.BlockSpec((br, 16), lambda i: (i, 0)),
            pl.BlockSpec((1, 256), lambda i: (0, 0)),
            pl.BlockSpec((256, 256), lambda i: (0, 0)),
            pl.BlockSpec((256, 16), lambda i: (0, 0)),
            pl.BlockSpec((256, 16), lambda i: (0, 0)),
            pl.BlockSpec((16, 16), lambda i: (0, 0)),
        ],
        out_specs=[
            pl.BlockSpec((br, 128), lambda i: (i, 0)),
            pl.BlockSpec((br, 128), lambda i: (i, 0)),
            pl.BlockSpec((br, 16), lambda i: (i, 0)),
            pl.BlockSpec((br, 16), lambda i: (i, 0)),
        ],
        out_shape=[
            jax.ShapeDtypeStruct((npad, 128), jnp.float32),
            jax.ShapeDtypeStruct((npad, 128), jnp.float32),
            jax.ShapeDtypeStruct((npad, 16), jnp.float32),
            jax.ShapeDtypeStruct((npad, 16), jnp.float32),
        ],
    )
    return f(xa, xb, sec_p, bias, W, A_src, A_dst, Wsec_p)


def _edge_body(est_hbm, edt_hbm, h0_hbm, h1_hbm, ei_hbm,
               alpha_hbm, agg0_hbm, agg1_hbm,
               idx2_0, idx2_1, idxp_0, idxp_1, idx_sc, idx_oc,
               arow0, brow0, dact0, arow1, brow1, dact1,
               exbuf, hrows, albuf, maxbuf, gall,
               denom_sh, out_sh, gmax_sh,
               sem_a0, sem_b0, sem_d0, sem_a1, sem_b1, sem_d1,
               sem_i0, sem_i1, sem_h, sem_sc, sem_out, sem_al):
    npad = agg0_hbm.shape[0]
    e_total = ei_hbm.shape[1]
    units = e_total // _CH
    q, rem = divmod(units, _NS)
    rows_per = npad // _NS
    c = lax.axis_index("c")
    s = lax.axis_index("s")
    nchunks = q + jnp.where(s < rem, 1, 0)
    ebase = (s * q + jnp.minimum(s, rem)) * _CH
    r0 = s * rows_per

    def _zero_tiles(i, _):
        exbuf[i, :] = jnp.zeros((_LANES,), jnp.float32)
        for j in range(8):
            hrows[i, pl.ds(16 * j, 16)] = jnp.zeros((_LANES,), jnp.float32)
        return 0
    lax.fori_loop(0, _CH, _zero_tiles, 0)
    for k in range(rows_per // _CH):
        pltpu.sync_copy(exbuf, denom_sh.at[pl.ds(r0 + k * _CH, _CH)])
        pltpu.sync_copy(hrows, out_sh.at[pl.ds(r0 + k * _CH, _CH)])

    def _max_chunks(hbm_ref, buf):
        def _chunk(k, m):
            pltpu.sync_copy(hbm_ref.at[pl.ds(r0 + k * _CH, _CH)], buf)
            def _mr(i, mm):
                return jnp.maximum(mm, buf[i, :])
            return lax.fori_loop(0, _CH, _mr, m, unroll=4)
        return lax.fori_loop(0, rows_per // _CH, _chunk,
                             jnp.zeros((_LANES,), jnp.float32))
    maxes = _max_chunks(est_hbm, arow0)
    maxed = _max_chunks(edt_hbm, brow0)
    maxbuf[0, :] = maxes
    pltpu.sync_copy(maxbuf, gmax_sh.at[pl.ds(s, 1)])
    maxbuf[0, :] = maxed
    pltpu.sync_copy(maxbuf, gmax_sh.at[pl.ds(_NS + s, 1)])
    plsc.subcore_barrier()

    pltpu.sync_copy(gmax_sh, gall)
    def _gmax_row(k, m):
        return jnp.maximum(m, gall[k, :])
    ges = lax.fori_loop(0, _NS, _gmax_row,
                        jnp.full((_LANES,), -3.0e38, jnp.float32))
    ged = lax.fori_loop(_NS, 2 * _NS, _gmax_row,
                        jnp.full((_LANES,), -3.0e38, jnp.float32))
    t = ges + ged
    gmax = jnp.where(t > 0, t, 0.2 * t)

    def _logit(k):
        t = arow[k, :] + brow[k, :]
        return jnp.where(t > 0, t, 0.2 * t)

    slot_idx = ((idx2_0.at[0], idx2_0.at[1]), (idx2_1.at[0], idx2_1.at[1]))
    slot_i2 = (idx2_0, idx2_1)
    slot_ip = ((idxp_0, sem_i0), (idxp_1, sem_i1))
    slot_buf = ((arow0, brow0, dact0, sem_a0, sem_b0, sem_d0),
                (arow1, brow1, dact1, sem_a1, sem_b1, sem_d1))

    def _load_idx(i, sl):
        base = ebase + i * _CH
        pltpu.sync_copy(ei_hbm.at[:, pl.ds(base, _CH)], slot_i2[sl])

    def _fire_idxp(i, sl):
        ip, si = slot_ip[sl]
        base = ebase + i * _CH
        pltpu.async_copy(ei_hbm.at[:, pl.ds(base, _CH)], ip, si)

    def _copy_idxp(i, sl):
        ip, si = slot_ip[sl]
        base = ebase + i * _CH
        pltpu.make_async_copy(ei_hbm.at[:, pl.ds(base, _CH)], ip, si).wait()
        i2 = slot_i2[sl]
        for r in range(2):
            for k in range(_CH // 16):
                i2[r, pl.ds(16 * k, 16)] = ip[r, pl.ds(16 * k, 16)]

    def _fire_ab(sl):
        isr, idr = slot_idx[sl]
        ar, br, _, sa, sb, _ = slot_buf[sl]
        pltpu.async_copy(est_hbm.at[isr], ar, sa)
        pltpu.async_copy(edt_hbm.at[idr], br, sb)

    def _fire_d(sl):
        _, idr = slot_idx[sl]
        _, _, da, _, _, sd = slot_buf[sl]
        pltpu.async_copy(denom_sh.at[idr], da, sd)

    def _wait_ab(sl):
        isr, idr = slot_idx[sl]
        ar, br, _, sa, sb, _ = slot_buf[sl]
        pltpu.make_async_copy(est_hbm.at[isr], ar, sa).wait()
        pltpu.make_async_copy(edt_hbm.at[idr], br, sb).wait()

    def _compute_ex(sl):
        ar, br = slot_buf[sl][0], slot_buf[sl][1]
        def _row(k, _2):
            t = ar[k, :] + br[k, :]
            l = jnp.where(t > 0, t, 0.2 * t)
            exbuf[k, :] = jnp.exp(l - gmax)
            return 0
        lax.fori_loop(0, _CH, _row, 0, unroll=4)

    def _snap_idx(sl, dst_ref):
        i2 = slot_i2[sl]
        for k in range(_CH // 16):
            dst_ref[pl.ds(16 * k, 16)] = i2[1, pl.ds(16 * k, 16)]

    def _b_step(i, sl, prefetch):
        if prefetch:
            @pl.when(i + 2 < q)
            def _():
                _fire_idxp(i + 2, sl)

        _wait_ab(sl)

        @pl.when(i > 0)
        def _():
            pltpu.make_async_copy(exbuf, denom_sh.at[idx_sc], sem_sc).wait()

        _compute_ex(sl)
        _snap_idx(sl, idx_sc)
        pltpu.async_copy(exbuf, denom_sh.at[idx_sc], sem_sc, add=True)
        if prefetch:
            @pl.when(i + 2 < q)
            def _():
                _copy_idxp(i + 2, sl)
                _fire_ab(sl)

    _load_idx(0, 0)
    _fire_ab(0)
    _load_idx(1, 1)
    _fire_ab(1)

    def _b_pair(i2, _):
        _b_step(2 * i2, 0, True)
        _b_step(2 * i2 + 1, 1, True)
        return 0
    lax.fori_loop(0, q // 2, _b_pair, 0)
    if q % 2:
        _b_step(q - 1, (q - 1) % 2, False)

    @pl.when(s < rem)
    def _():
        _load_idx(q, 0)
        _fire_ab(0)
        _b_step(q, 0, False)

    pltpu.make_async_copy(exbuf, denom_sh.at[idx_sc], sem_sc).wait()
    plsc.subcore_barrier()

    def _recip_chunk(k, _):
        off = r0 + k * _CH
        pltpu.sync_copy(denom_sh.at[pl.ds(off, _CH)], dact0)
        def _rr(i, _2):
            dact0[i, :] = 1.0 / (dact0[i, :] + 1e-9)
            return 0
        lax.fori_loop(0, _CH, _rr, 0, unroll=4)
        pltpu.sync_copy(dact0, denom_sh.at[pl.ds(off, _CH)])
        return 0
    lax.fori_loop(0, rows_per // _CH, _recip_chunk, 0)
    plsc.subcore_barrier()

    write_alpha = jnp.logical_or(
        jnp.logical_and(c == 0, s < _NS // 2),
        jnp.logical_and(c == 1, s >= _NS // 2))

    def _c_step(i, sl, prefetch):
        isr, idr = slot_idx[sl]
        da, sd = slot_buf[sl][2], slot_buf[sl][5]
        base = ebase + i * _CH

        if prefetch:
            @pl.when(i + 2 < q)
            def _():
                _fire_idxp(i + 2, sl)

        @pl.when(i > 0)
        def _():
            pltpu.make_async_copy(hrows, out_sh.at[idx_oc], sem_out).wait()

        @pl.when(c == 0)
        def _():
            pltpu.async_copy(h0_hbm.at[isr], hrows, sem_h)

        @pl.when(c == 1)
        def _():
            pltpu.async_copy(h1_hbm.at[isr], hrows, sem_h)

        @pl.when(jnp.logical_and(write_alpha, i > 0))
        def _():
            pltpu.make_async_copy(
                albuf, alpha_hbm.at[pl.ds(0, _CH // 8)], sem_al).wait()

        _wait_ab(sl)
        _compute_ex(sl)
        pltpu.make_async_copy(denom_sh.at[idr], da, sd).wait()
        def _row2(k, _2):
            a = exbuf[k, :] * da[k, :]
            exbuf[k, :] = a
            albuf[k // 8, pl.ds(16 * (k % 8), 16)] = a
            return 0
        lax.fori_loop(0, _CH, _row2, 0, unroll=4)

        def _scale_rows(hc):
            def _scale(k, _2):
                av = exbuf[k, :]
                for j in range(8):
                    a = av[hc + (j // 2)]
                    hrows[k, pl.ds(16 * j, 16)] = hrows[k, pl.ds(16 * j, 16)] * a
                return 0
            lax.fori_loop(0, _CH, _scale, 0, unroll=2)

        @pl.when(write_alpha)
        def _():
            pbase = pl.multiple_of(base // 8, 8)
            pltpu.async_copy(albuf, alpha_hbm.at[pl.ds(pbase, _CH // 8)],
                             sem_al)

        @pl.when(c == 0)
        def _():
            pltpu.make_async_copy(h0_hbm.at[isr], hrows, sem_h).wait()
            _scale_rows(0)

        @pl.when(c == 1)
        def _():
            pltpu.make_async_copy(h1_hbm.at[isr], hrows, sem_h).wait()
            _scale_rows(4)

        _snap_idx(sl, idx_oc)
        pltpu.async_copy(hrows, out_sh.at[idx_oc], sem_out, add=True)
        if prefetch:
            @pl.when(i + 2 < q)
            def _():
                _copy_idxp(i + 2, sl)
                _fire_ab(sl)
                _fire_d(sl)

    _load_idx(0, 0)
    _fire_ab(0)
    _fire_d(0)
    _load_idx(1, 1)
    _fire_ab(1)
    _fire_d(1)

    def _c_pair(i2, _):
        _c_step(2 * i2, 0, True)
        _c_step(2 * i2 + 1, 1, True)
        return 0
    lax.fori_loop(0, q // 2, _c_pair, 0)
    if q % 2:
        _c_step(q - 1, (q - 1) % 2, False)

    @pl.when(s < rem)
    def _():
        _load_idx(q, 0)
        _fire_ab(0)
        _fire_d(0)
        _c_step(q, 0, False)

    pltpu.make_async_copy(hrows, out_sh.at[idx_oc], sem_out).wait()

    @pl.when(write_alpha)
    def _():
        pltpu.make_async_copy(albuf, alpha_hbm.at[pl.ds(0, _CH // 8)],
                              sem_al).wait()
    plsc.subcore_barrier()

    @pl.when(c == 0)
    def _():
        pltpu.sync_copy(out_sh.at[pl.ds(r0, rows_per)],
                        agg0_hbm.at[pl.ds(r0, rows_per)])

    @pl.when(c == 1)
    def _():
        pltpu.sync_copy(out_sh.at[pl.ds(r0, rows_per)],
                        agg1_hbm.at[pl.ds(r0, rows_per)])


def _edge_phase(est, edt, h0, h1, ei):
    npad = est.shape[0]
    e_total = ei.shape[1]
    mesh = plsc.VectorSubcoreMesh(core_axis_name="c", subcore_axis_name="s")
    f = pl.kernel(
        _edge_body,
        out_type=(jax.ShapeDtypeStruct((e_total // 8, 128), jnp.float32),
                  jax.ShapeDtypeStruct((npad, 128), jnp.float32),
                  jax.ShapeDtypeStruct((npad, 128), jnp.float32)),
        mesh=mesh,
        scratch_types=(
            pltpu.VMEM((2, _CH), jnp.int32),
            pltpu.VMEM((2, _CH), jnp.int32),
            pltpu.VMEM((2, _CH), jnp.int32),
            pltpu.VMEM((2, _CH), jnp.int32),
            pltpu.VMEM((_CH,), jnp.int32),
            pltpu.VMEM((_CH,), jnp.int32),
            pltpu.VMEM((_CH, 16), jnp.float32),
            pltpu.VMEM((_CH, 16), jnp.float32),
            pltpu.VMEM((_CH, 16), jnp.float32),
            pltpu.VMEM((_CH, 16), jnp.float32),
            pltpu.VMEM((_CH, 16), jnp.float32),
            pltpu.VMEM((_CH, 16), jnp.float32),
            pltpu.VMEM((_CH, 16), jnp.float32),
            pltpu.VMEM((_CH, 128), jnp.float32),
            pltpu.VMEM((_CH // 8, 128), jnp.float32),
            pltpu.VMEM((1, 16), jnp.float32),
            pltpu.VMEM((2 * _NS, 16), jnp.float32),
            pltpu.VMEM_SHARED((npad, 16), jnp.float32),
            pltpu.VMEM_SHARED((npad, 128), jnp.float32),
            pltpu.VMEM_SHARED((2 * _NS, 16), jnp.float32),
            pltpu.SemaphoreType.DMA,
            pltpu.SemaphoreType.DMA,
            pltpu.SemaphoreType.DMA,
            pltpu.SemaphoreType.DMA,
            pltpu.SemaphoreType.DMA,
            pltpu.SemaphoreType.DMA,
            pltpu.SemaphoreType.DMA,
            pltpu.SemaphoreType.DMA,
            pltpu.SemaphoreType.DMA,
            pltpu.SemaphoreType.DMA,
            pltpu.SemaphoreType.DMA,
            pltpu.SemaphoreType.DMA,
        ),
        compiler_params=pltpu.CompilerParams(use_tc_tiling_on_sc=False),
    )
    return f(est, edt, h0, h1, ei)


def _conv_kernel(a0c_ref, a1c_ref, a0p_ref, a1p_ref, bias_ref,
                 w10_ref, w11_ref, w12_ref, b1_ref,
                 w20_ref, w21_ref, w22_ref, b2_ref, y_ref, *, br):
    i = pl.program_id(0)

    def _elu(t):
        return jnp.where(t > 0, t, jnp.exp(jnp.minimum(t, 0.0)) - 1.0)

    xc = _elu(jnp.concatenate([a0c_ref[...], a1c_ref[...]], axis=1)
              + bias_ref[...])
    xp = _elu(jnp.concatenate([a0p_ref[...], a1p_ref[...]], axis=1)
              + bias_ref[...])
    halo = jnp.where(i == 0, 0.0, xp[br - 6:, :])
    xw = jnp.concatenate([halo, xc], axis=0)

    y1 = (jnp.dot(xw[2:br + 6], w12_ref[...], preferred_element_type=jnp.float32)
          + jnp.dot(xw[1:br + 5], w11_ref[...], preferred_element_type=jnp.float32)
          + jnp.dot(xw[0:br + 4], w10_ref[...], preferred_element_type=jnp.float32)
          + b1_ref[...])
    y1 = jnp.maximum(y1, 0.0)
    rows = lax.broadcasted_iota(jnp.int32, (br + 4, 1), 0) + i * br - 4
    y1 = jnp.where(rows >= 0, y1, 0.0)

    y = (jnp.dot(y1[4:br + 4], w22_ref[...], preferred_element_type=jnp.float32)
         + jnp.dot(y1[2:br + 2], w21_ref[...], preferred_element_type=jnp.float32)
         + jnp.dot(y1[0:br], w20_ref[...], preferred_element_type=jnp.float32)
         + b2_ref[...])
    y_ref[...] = jnp.maximum(y, 0.0)


def _convs(a0, a1, bias, conv1_w, conv1_b, conv2_w, conv2_b, n_out):
    br = 200
    grid = (n_out // br,)
    prev = lambda i: (jnp.maximum(i - 1, 0), 0)
    f = pl.pallas_call(
        functools.partial(_conv_kernel, br=br),
        grid=grid,
        in_specs=[
            pl.BlockSpec((br, 128), lambda i: (i, 0)),
            pl.BlockSpec((br, 128), lambda i: (i, 0)),
            pl.BlockSpec((br, 128), prev),
            pl.BlockSpec((br, 128), prev),
            pl.BlockSpec((1, 256), lambda i: (0, 0)),
            pl.BlockSpec((256, 256), lambda i: (0, 0)),
            pl.BlockSpec((256, 256), lambda i: (0, 0)),
            pl.BlockSpec((256, 256), lambda i: (0, 0)),
            pl.BlockSpec((1, 256), lambda i: (0, 0)),
            pl.BlockSpec((256, 256), lambda i: (0, 0)),
            pl.BlockSpec((256, 256), lambda i: (0, 0)),
            pl.BlockSpec((256, 256), lambda i: (0, 0)),
            pl.BlockSpec((1, 256), lambda i: (0, 0)),
        ],
        out_specs=pl.BlockSpec((br, 256), lambda i: (i, 0)),
        out_shape=jax.ShapeDtypeStruct((n_out, 256), jnp.float32),
    )
    return f(a0, a1, a0, a1, bias,
             conv1_w[0], conv1_w[1], conv1_w[2], conv1_b.reshape(1, 256),
             conv2_w[0], conv2_w[1], conv2_w[2], conv2_b.reshape(1, 256))


def _head_proj(a):
    h_heads, hdim = a.shape
    eye = jnp.eye(h_heads, dtype=a.dtype)
    blk = (a[:, :, None] * eye[:, None, :]).reshape(h_heads * hdim, h_heads)
    return jnp.concatenate(
        [blk, jnp.zeros((h_heads * hdim, 16 - h_heads), a.dtype)], axis=1)


def _pad_wsec(wsec):
    sec, h_heads = wsec.shape
    return jnp.concatenate(
        [wsec, jnp.zeros((sec, 16 - h_heads), wsec.dtype)], axis=1)


def kernel(entities, actions, edge_index, security_features, entity_emb,
           action_emb, W1, a_src1, a_dst1, Wsec1, b1, W2, a_src2, a_dst2,
           Wsec2, b2, conv1_w, conv1_b, conv2_w, conv2_b):
    n = entities.shape[0]
    npad = _ceil_to(n, max(_NW * _CHE, _NS * _CH))

    ent_p = jnp.concatenate(
        [entities.astype(jnp.int32), jnp.zeros((npad - n,), jnp.int32)])
    act_p = jnp.concatenate(
        [actions.astype(jnp.int32), jnp.zeros((npad - n,), jnp.int32)])
    sec_p = jnp.concatenate(
        [security_features,
         jnp.zeros((npad - n, security_features.shape[1]), jnp.float32)])
    ei = edge_index.astype(jnp.int32)

    xe, xa = _embed_gather(ent_p, act_p, entity_emb, action_emb, npad)

    zero_b = jnp.zeros((1, 256), jnp.float32)
    h0, h1, est, edt = _proj(xe, xa, sec_p, zero_b, W1, _head_proj(a_src1),
                             _head_proj(a_dst1), _pad_wsec(Wsec1),
                             apply_elu=False)
    alpha1_p, g0, g1 = _edge_phase(est, edt, h0, h1, ei)

    h0b, h1b, est2, edt2 = _proj(g0, g1, sec_p, b1.reshape(1, 256), W2,
                                 _head_proj(a_src2), _head_proj(a_dst2),
                                 _pad_wsec(Wsec2), apply_elu=True)
    alpha2_p, f0, f1 = _edge_phase(est2, edt2, h0b, h1b, ei)

    y = _convs(f0, f1, b2.reshape(1, 256), conv1_w, conv1_b, conv2_w, conv2_b,
               n_out=n)
    e_total = ei.shape[1]
    att1 = alpha1_p.reshape(e_total, 16)[:, :8]
    att2 = alpha2_p.reshape(e_total, 16)[:, :8]
    return y, att1, att2


# speedup vs baseline: 1.4925x; 1.4925x over previous
import functools

import jax
import jax.numpy as jnp
from jax import lax
from jax.experimental import pallas as pl
from jax.experimental.pallas import tpu as pltpu
from jax.experimental.pallas import tpu_sc as plsc

_NC = 2
_NS = 16
_LANES = 16
_NW = _NC * _NS
_CH = 128
_CHE = 64


def _ceil_to(x, m):
    return -(-x // m) * m


def _embed_body(ent_hbm, act_hbm, eemb_hbm, aemb_hbm, xe_hbm, xa_hbm,
                ie0, ia0, ie1, ia1, re0, ra0, re1, ra1,
                se0, sa0, se1, sa1, swe, swa):
    npad = xe_hbm.shape[0]
    per_w = npad // _NW
    w = lax.axis_index("s") * _NC + lax.axis_index("c")
    base_w = w * per_w
    n_it = per_w // _CHE
    idxs = ((ie0, ia0), (ie1, ia1))
    rows = ((re0, ra0), (re1, ra1))
    sems = ((se0, sa0), (se1, sa1))

    def _load(k, sl):
        base = base_w + k * _CHE
        pltpu.sync_copy(ent_hbm.at[pl.ds(base, _CHE)], idxs[sl][0])
        pltpu.sync_copy(act_hbm.at[pl.ds(base, _CHE)], idxs[sl][1])

    def _fire(sl):
        pltpu.async_copy(eemb_hbm.at[idxs[sl][0]], rows[sl][0], sems[sl][0])
        pltpu.async_copy(aemb_hbm.at[idxs[sl][1]], rows[sl][1], sems[sl][1])

    _load(0, 0)
    _fire(0)
    if n_it > 1:
        _load(1, 1)
        _fire(1)
    for k in range(n_it):
        sl = k % 2
        base = base_w + k * _CHE
        pltpu.make_async_copy(eemb_hbm.at[idxs[sl][0]], rows[sl][0],
                              sems[sl][0]).wait()
        pltpu.make_async_copy(aemb_hbm.at[idxs[sl][1]], rows[sl][1],
                              sems[sl][1]).wait()
        pltpu.async_copy(rows[sl][0], xe_hbm.at[pl.ds(base, _CHE)], swe)
        pltpu.async_copy(rows[sl][1], xa_hbm.at[pl.ds(base, _CHE)], swa)
        if k + 2 < n_it:
            _load(k + 2, sl)
            pltpu.make_async_copy(rows[sl][0], xe_hbm.at[pl.ds(base, _CHE)],
                                  swe).wait()
            pltpu.make_async_copy(rows[sl][1], xa_hbm.at[pl.ds(base, _CHE)],
                                  swa).wait()
            _fire(sl)
    for k in (n_it - 2, n_it - 1):
        if k < 0:
            continue
        sl = k % 2
        base = base_w + k * _CHE
        pltpu.make_async_copy(rows[sl][0], xe_hbm.at[pl.ds(base, _CHE)],
                              swe).wait()
        pltpu.make_async_copy(rows[sl][1], xa_hbm.at[pl.ds(base, _CHE)],
                              swa).wait()


def _embed_gather(ent_p, act_p, entity_emb, action_emb, npad):
    mesh = plsc.VectorSubcoreMesh(core_axis_name="c", subcore_axis_name="s")
    f = pl.kernel(
        _embed_body,
        out_type=(jax.ShapeDtypeStruct((npad, 128), jnp.float32),
                  jax.ShapeDtypeStruct((npad, 128), jnp.float32)),
        mesh=mesh,
        scratch_types=(
            pltpu.VMEM((_CHE,), jnp.int32),
            pltpu.VMEM((_CHE,), jnp.int32),
            pltpu.VMEM((_CHE,), jnp.int32),
            pltpu.VMEM((_CHE,), jnp.int32),
            pltpu.VMEM((_CHE, 128), jnp.float32),
            pltpu.VMEM((_CHE, 128), jnp.float32),
            pltpu.VMEM((_CHE, 128), jnp.float32),
            pltpu.VMEM((_CHE, 128), jnp.float32),
            pltpu.SemaphoreType.DMA,
            pltpu.SemaphoreType.DMA,
            pltpu.SemaphoreType.DMA,
            pltpu.SemaphoreType.DMA,
            pltpu.SemaphoreType.DMA,
            pltpu.SemaphoreType.DMA,
        ),
    )
    return f(ent_p, act_p, entity_emb, action_emb)


def _proj_kernel(xa_ref, xb_ref, sec_ref, bias_ref, w_ref, asrc_ref, adst_ref,
                 wsec_ref, h0_ref, h1_ref, est_ref, edt_ref, *, apply_elu):
    x = jnp.concatenate([xa_ref[...], xb_ref[...]], axis=1)
    if apply_elu:
        t = x + bias_ref[...]
        x = jnp.where(t > 0, t, jnp.exp(jnp.minimum(t, 0.0)) - 1.0)
    h = jnp.dot(x, w_ref[...], preferred_element_type=jnp.float32)
    h0_ref[...] = h[:, :128]
    h1_ref[...] = h[:, 128:]
    est_ref[...] = jnp.dot(h, asrc_ref[...], preferred_element_type=jnp.float32)
    edt_ref[...] = (jnp.dot(h, adst_ref[...], preferred_element_type=jnp.float32)
                    + jnp.dot(sec_ref[...], wsec_ref[...],
                              preferred_element_type=jnp.float32))


def _proj(xa, xb, sec_p, bias, W, A_src, A_dst, Wsec_p, apply_elu):
    npad = xa.shape[0]
    br = 512
    grid = (npad // br,)
    f = pl.pallas_call(
        functools.partial(_proj_kernel, apply_elu=apply_elu),
        grid=grid,
        in_specs=[
            pl.BlockSpec((br, 128), lambda i: (i, 0)),
            pl.BlockSpec((br, 128), lambda i: (i, 0)),
            pl.BlockSpec((br, 16), lambda i: (i, 0)),
            pl.BlockSpec((1, 256), lambda i: (0, 0)),
            pl.BlockSpec((256, 256), lambda i: (0, 0)),
            pl.BlockSpec((256, 16), lambda i: (0, 0)),
            pl.BlockSpec((256, 16), lambda i: (0, 0)),
            pl.BlockSpec((16, 16), lambda i: (0, 0)),
        ],
        out_specs=[
            pl.BlockSpec((br, 128), lambda i: (i, 0)),
            pl.BlockSpec((br, 128), lambda i: (i, 0)),
            pl.BlockSpec((br, 16), lambda i: (i, 0)),
            pl.BlockSpec((br, 16), lambda i: (i, 0)),
        ],
        out_shape=[
            jax.ShapeDtypeStruct((npad, 128), jnp.float32),
            jax.ShapeDtypeStruct((npad, 128), jnp.float32),
            jax.ShapeDtypeStruct((npad, 16), jnp.float32),
            jax.ShapeDtypeStruct((npad, 16), jnp.float32),
        ],
    )
    return f(xa, xb, sec_p, bias, W, A_src, A_dst, Wsec_p)


def _edge_body(est_hbm, edt_hbm, h0_hbm, h1_hbm, ei_hbm,
               alpha_hbm, agg0_hbm, agg1_hbm,
               idx2_0, idx2_1, idxp_0, idxp_1, idx_sc, idx_oc,
               arow0, brow0, dact0, arow1, brow1, dact1,
               exbuf, hrows, albuf, maxbuf, gall,
               denom_sh, out_sh, gmax_sh,
               sem_a0, sem_b0, sem_d0, sem_a1, sem_b1, sem_d1,
               sem_i0, sem_i1, sem_h, sem_sc, sem_out, sem_al):
    npad = agg0_hbm.shape[0]
    e_total = ei_hbm.shape[1]
    units = e_total // _CH
    q, rem = divmod(units, _NS)
    rows_per = npad // _NS
    c = lax.axis_index("c")
    s = lax.axis_index("s")
    nchunks = q + jnp.where(s < rem, 1, 0)
    ebase = (s * q + jnp.minimum(s, rem)) * _CH
    r0 = s * rows_per

    def _zero_tiles(i, _):
        exbuf[i, :] = jnp.zeros((_LANES,), jnp.float32)
        for j in range(8):
            hrows[i, pl.ds(16 * j, 16)] = jnp.zeros((_LANES,), jnp.float32)
        return 0
    lax.fori_loop(0, _CH, _zero_tiles, 0)
    for k in range(rows_per // _CH):
        pltpu.sync_copy(exbuf, denom_sh.at[pl.ds(r0 + k * _CH, _CH)])
        pltpu.sync_copy(hrows, out_sh.at[pl.ds(r0 + k * _CH, _CH)])

    def _max_chunks(hbm_ref, buf):
        def _chunk(k, m):
            pltpu.sync_copy(hbm_ref.at[pl.ds(r0 + k * _CH, _CH)], buf)
            def _mr(i, mm):
                return jnp.maximum(mm, buf[i, :])
            return lax.fori_loop(0, _CH, _mr, m)
        return lax.fori_loop(0, rows_per // _CH, _chunk,
                             jnp.zeros((_LANES,), jnp.float32))
    maxes = _max_chunks(est_hbm, arow0)
    maxed = _max_chunks(edt_hbm, brow0)
    maxbuf[0, :] = maxes
    pltpu.sync_copy(maxbuf, gmax_sh.at[pl.ds(s, 1)])
    maxbuf[0, :] = maxed
    pltpu.sync_copy(maxbuf, gmax_sh.at[pl.ds(_NS + s, 1)])
    plsc.subcore_barrier()

    pltpu.sync_copy(gmax_sh, gall)
    def _gmax_row(k, m):
        return jnp.maximum(m, gall[k, :])
    ges = lax.fori_loop(0, _NS, _gmax_row,
                        jnp.full((_LANES,), -3.0e38, jnp.float32))
    ged = lax.fori_loop(_NS, 2 * _NS, _gmax_row,
                        jnp.full((_LANES,), -3.0e38, jnp.float32))
    t = ges + ged
    gmax = jnp.where(t > 0, t, 0.2 * t)

    def _logit(k):
        t = arow[k, :] + brow[k, :]
        return jnp.where(t > 0, t, 0.2 * t)

    slot_idx = ((idx2_0.at[0], idx2_0.at[1]), (idx2_1.at[0], idx2_1.at[1]))
    slot_i2 = (idx2_0, idx2_1)
    slot_ip = ((idxp_0, sem_i0), (idxp_1, sem_i1))
    slot_buf = ((arow0, brow0, dact0, sem_a0, sem_b0, sem_d0),
                (arow1, brow1, dact1, sem_a1, sem_b1, sem_d1))

    def _load_idx(i, sl):
        base = ebase + i * _CH
        pltpu.sync_copy(ei_hbm.at[:, pl.ds(base, _CH)], slot_i2[sl])

    def _fire_idxp(i, sl):
        ip, si = slot_ip[sl]
        base = ebase + i * _CH
        pltpu.async_copy(ei_hbm.at[:, pl.ds(base, _CH)], ip, si)

    def _copy_idxp(i, sl):
        ip, si = slot_ip[sl]
        base = ebase + i * _CH
        pltpu.make_async_copy(ei_hbm.at[:, pl.ds(base, _CH)], ip, si).wait()
        i2 = slot_i2[sl]
        for r in range(2):
            for k in range(_CH // 16):
                i2[r, pl.ds(16 * k, 16)] = ip[r, pl.ds(16 * k, 16)]

    def _fire_ab(sl):
        isr, idr = slot_idx[sl]
        ar, br, _, sa, sb, _ = slot_buf[sl]
        pltpu.async_copy(est_hbm.at[isr], ar, sa)
        pltpu.async_copy(edt_hbm.at[idr], br, sb)

    def _fire_d(sl):
        _, idr = slot_idx[sl]
        _, _, da, _, _, sd = slot_buf[sl]
        pltpu.async_copy(denom_sh.at[idr], da, sd)

    def _wait_ab(sl):
        isr, idr = slot_idx[sl]
        ar, br, _, sa, sb, _ = slot_buf[sl]
        pltpu.make_async_copy(est_hbm.at[isr], ar, sa).wait()
        pltpu.make_async_copy(edt_hbm.at[idr], br, sb).wait()

    def _compute_ex(sl):
        ar, br = slot_buf[sl][0], slot_buf[sl][1]
        def _row(k, _2):
            t = ar[k, :] + br[k, :]
            l = jnp.where(t > 0, t, 0.2 * t)
            exbuf[k, :] = jnp.exp(l - gmax)
            return 0
        lax.fori_loop(0, _CH, _row, 0)

    def _snap_idx(sl, dst_ref):
        i2 = slot_i2[sl]
        for k in range(_CH // 16):
            dst_ref[pl.ds(16 * k, 16)] = i2[1, pl.ds(16 * k, 16)]

    def _b_step(i, sl, prefetch):
        if prefetch:
            @pl.when(i + 2 < q)
            def _():
                _fire_idxp(i + 2, sl)

        _wait_ab(sl)

        @pl.when(i > 0)
        def _():
            pltpu.make_async_copy(exbuf, denom_sh.at[idx_sc], sem_sc).wait()

        _compute_ex(sl)
        _snap_idx(sl, idx_sc)
        pltpu.async_copy(exbuf, denom_sh.at[idx_sc], sem_sc, add=True)
        if prefetch:
            @pl.when(i + 2 < q)
            def _():
                _copy_idxp(i + 2, sl)
                _fire_ab(sl)

    _load_idx(0, 0)
    _fire_ab(0)
    _load_idx(1, 1)
    _fire_ab(1)

    def _b_pair(i2, _):
        _b_step(2 * i2, 0, True)
        _b_step(2 * i2 + 1, 1, True)
        return 0
    lax.fori_loop(0, q // 2, _b_pair, 0)
    if q % 2:
        _b_step(q - 1, (q - 1) % 2, False)

    @pl.when(s < rem)
    def _():
        _load_idx(q, 0)
        _fire_ab(0)
        _b_step(q, 0, False)

    pltpu.make_async_copy(exbuf, denom_sh.at[idx_sc], sem_sc).wait()
    plsc.subcore_barrier()

    def _recip_chunk(k, _):
        off = r0 + k * _CH
        pltpu.sync_copy(denom_sh.at[pl.ds(off, _CH)], dact0)
        def _rr(i, _2):
            dact0[i, :] = 1.0 / (dact0[i, :] + 1e-9)
            return 0
        lax.fori_loop(0, _CH, _rr, 0)
        pltpu.sync_copy(dact0, denom_sh.at[pl.ds(off, _CH)])
        return 0
    lax.fori_loop(0, rows_per // _CH, _recip_chunk, 0)
    plsc.subcore_barrier()

    write_alpha = jnp.logical_or(
        jnp.logical_and(c == 0, s < _NS // 2),
        jnp.logical_and(c == 1, s >= _NS // 2))

    def _c_step(i, sl, prefetch):
        isr, idr = slot_idx[sl]
        da, sd = slot_buf[sl][2], slot_buf[sl][5]
        base = ebase + i * _CH

        if prefetch:
            @pl.when(i + 2 < q)
            def _():
                _fire_idxp(i + 2, sl)

        @pl.when(i > 0)
        def _():
            pltpu.make_async_copy(hrows, out_sh.at[idx_oc], sem_out).wait()

        @pl.when(c == 0)
        def _():
            pltpu.async_copy(h0_hbm.at[isr], hrows, sem_h)

        @pl.when(c == 1)
        def _():
            pltpu.async_copy(h1_hbm.at[isr], hrows, sem_h)

        @pl.when(jnp.logical_and(write_alpha, i > 0))
        def _():
            pltpu.make_async_copy(
                albuf, alpha_hbm.at[pl.ds(0, _CH // 8)], sem_al).wait()

        _wait_ab(sl)
        _compute_ex(sl)
        pltpu.make_async_copy(denom_sh.at[idr], da, sd).wait()
        def _row2(k, _2):
            a = exbuf[k, :] * da[k, :]
            exbuf[k, :] = a
            albuf[k // 8, pl.ds(16 * (k % 8), 16)] = a
            return 0
        lax.fori_loop(0, _CH, _row2, 0)

        def _scale_rows(hc):
            def _scale(k, _2):
                av = exbuf[k, :]
                for j in range(8):
                    a = av[hc + (j // 2)]
                    hrows[k, pl.ds(16 * j, 16)] = hrows[k, pl.ds(16 * j, 16)] * a
                return 0
            lax.fori_loop(0, _CH, _scale, 0)

        @pl.when(write_alpha)
        def _():
            pbase = pl.multiple_of(base // 8, 8)
            pltpu.async_copy(albuf, alpha_hbm.at[pl.ds(pbase, _CH // 8)],
                             sem_al)

        @pl.when(c == 0)
        def _():
            pltpu.make_async_copy(h0_hbm.at[isr], hrows, sem_h).wait()
            _scale_rows(0)

        @pl.when(c == 1)
        def _():
            pltpu.make_async_copy(h1_hbm.at[isr], hrows, sem_h).wait()
            _scale_rows(4)

        _snap_idx(sl, idx_oc)
        pltpu.async_copy(hrows, out_sh.at[idx_oc], sem_out, add=True)
        if prefetch:
            @pl.when(i + 2 < q)
            def _():
                _copy_idxp(i + 2, sl)
                _fire_ab(sl)
                _fire_d(sl)

    _load_idx(0, 0)
    _fire_ab(0)
    _fire_d(0)
    _load_idx(1, 1)
    _fire_ab(1)
    _fire_d(1)

    def _c_pair(i2, _):
        _c_step(2 * i2, 0, True)
        _c_step(2 * i2 + 1, 1, True)
        return 0
    lax.fori_loop(0, q // 2, _c_pair, 0)
    if q % 2:
        _c_step(q - 1, (q - 1) % 2, False)

    @pl.when(s < rem)
    def _():
        _load_idx(q, 0)
        _fire_ab(0)
        _fire_d(0)
        _c_step(q, 0, False)

    pltpu.make_async_copy(hrows, out_sh.at[idx_oc], sem_out).wait()

    @pl.when(write_alpha)
    def _():
        pltpu.make_async_copy(albuf, alpha_hbm.at[pl.ds(0, _CH // 8)],
                              sem_al).wait()
    plsc.subcore_barrier()

    @pl.when(c == 0)
    def _():
        pltpu.sync_copy(out_sh.at[pl.ds(r0, rows_per)],
                        agg0_hbm.at[pl.ds(r0, rows_per)])

    @pl.when(c == 1)
    def _():
        pltpu.sync_copy(out_sh.at[pl.ds(r0, rows_per)],
                        agg1_hbm.at[pl.ds(r0, rows_per)])


def _edge_phase(est, edt, h0, h1, ei):
    npad = est.shape[0]
    e_total = ei.shape[1]
    mesh = plsc.VectorSubcoreMesh(core_axis_name="c", subcore_axis_name="s")
    f = pl.kernel(
        _edge_body,
        out_type=(jax.ShapeDtypeStruct((e_total // 8, 128), jnp.float32),
                  jax.ShapeDtypeStruct((npad, 128), jnp.float32),
                  jax.ShapeDtypeStruct((npad, 128), jnp.float32)),
        mesh=mesh,
        scratch_types=(
            pltpu.VMEM((2, _CH), jnp.int32),
            pltpu.VMEM((2, _CH), jnp.int32),
            pltpu.VMEM((2, _CH), jnp.int32),
            pltpu.VMEM((2, _CH), jnp.int32),
            pltpu.VMEM((_CH,), jnp.int32),
            pltpu.VMEM((_CH,), jnp.int32),
            pltpu.VMEM((_CH, 16), jnp.float32),
            pltpu.VMEM((_CH, 16), jnp.float32),
            pltpu.VMEM((_CH, 16), jnp.float32),
            pltpu.VMEM((_CH, 16), jnp.float32),
            pltpu.VMEM((_CH, 16), jnp.float32),
            pltpu.VMEM((_CH, 16), jnp.float32),
            pltpu.VMEM((_CH, 16), jnp.float32),
            pltpu.VMEM((_CH, 128), jnp.float32),
            pltpu.VMEM((_CH // 8, 128), jnp.float32),
            pltpu.VMEM((1, 16), jnp.float32),
            pltpu.VMEM((2 * _NS, 16), jnp.float32),
            pltpu.VMEM_SHARED((npad, 16), jnp.float32),
            pltpu.VMEM_SHARED((npad, 128), jnp.float32),
            pltpu.VMEM_SHARED((2 * _NS, 16), jnp.float32),
            pltpu.SemaphoreType.DMA,
            pltpu.SemaphoreType.DMA,
            pltpu.SemaphoreType.DMA,
            pltpu.SemaphoreType.DMA,
            pltpu.SemaphoreType.DMA,
            pltpu.SemaphoreType.DMA,
            pltpu.SemaphoreType.DMA,
            pltpu.SemaphoreType.DMA,
            pltpu.SemaphoreType.DMA,
            pltpu.SemaphoreType.DMA,
            pltpu.SemaphoreType.DMA,
            pltpu.SemaphoreType.DMA,
        ),
        compiler_params=pltpu.CompilerParams(use_tc_tiling_on_sc=False),
    )
    return f(est, edt, h0, h1, ei)


def _conv_kernel(a0c_ref, a1c_ref, a0p_ref, a1p_ref, bias_ref,
                 w10_ref, w11_ref, w12_ref, b1_ref,
                 w20_ref, w21_ref, w22_ref, b2_ref, y_ref, *, br):
    i = pl.program_id(0)

    def _elu(t):
        return jnp.where(t > 0, t, jnp.exp(jnp.minimum(t, 0.0)) - 1.0)

    xc = _elu(jnp.concatenate([a0c_ref[...], a1c_ref[...]], axis=1)
              + bias_ref[...])
    xp = _elu(jnp.concatenate([a0p_ref[...], a1p_ref[...]], axis=1)
              + bias_ref[...])
    halo = jnp.where(i == 0, 0.0, xp[br - 6:, :])
    xw = jnp.concatenate([halo, xc], axis=0)

    y1 = (jnp.dot(xw[2:br + 6], w12_ref[...], preferred_element_type=jnp.float32)
          + jnp.dot(xw[1:br + 5], w11_ref[...], preferred_element_type=jnp.float32)
          + jnp.dot(xw[0:br + 4], w10_ref[...], preferred_element_type=jnp.float32)
          + b1_ref[...])
    y1 = jnp.maximum(y1, 0.0)
    rows = lax.broadcasted_iota(jnp.int32, (br + 4, 1), 0) + i * br - 4
    y1 = jnp.where(rows >= 0, y1, 0.0)

    y = (jnp.dot(y1[4:br + 4], w22_ref[...], preferred_element_type=jnp.float32)
         + jnp.dot(y1[2:br + 2], w21_ref[...], preferred_element_type=jnp.float32)
         + jnp.dot(y1[0:br], w20_ref[...], preferred_element_type=jnp.float32)
         + b2_ref[...])
    y_ref[...] = jnp.maximum(y, 0.0)


def _convs(a0, a1, bias, conv1_w, conv1_b, conv2_w, conv2_b, n_out):
    br = 200
    grid = (n_out // br,)
    prev = lambda i: (jnp.maximum(i - 1, 0), 0)
    f = pl.pallas_call(
        functools.partial(_conv_kernel, br=br),
        grid=grid,
        in_specs=[
            pl.BlockSpec((br, 128), lambda i: (i, 0)),
            pl.BlockSpec((br, 128), lambda i: (i, 0)),
            pl.BlockSpec((br, 128), prev),
            pl.BlockSpec((br, 128), prev),
            pl.BlockSpec((1, 256), lambda i: (0, 0)),
            pl.BlockSpec((256, 256), lambda i: (0, 0)),
            pl.BlockSpec((256, 256), lambda i: (0, 0)),
            pl.BlockSpec((256, 256), lambda i: (0, 0)),
            pl.BlockSpec((1, 256), lambda i: (0, 0)),
            pl.BlockSpec((256, 256), lambda i: (0, 0)),
            pl.BlockSpec((256, 256), lambda i: (0, 0)),
            pl.BlockSpec((256, 256), lambda i: (0, 0)),
            pl.BlockSpec((1, 256), lambda i: (0, 0)),
        ],
        out_specs=pl.BlockSpec((br, 256), lambda i: (i, 0)),
        out_shape=jax.ShapeDtypeStruct((n_out, 256), jnp.float32),
    )
    return f(a0, a1, a0, a1, bias,
             conv1_w[0], conv1_w[1], conv1_w[2], conv1_b.reshape(1, 256),
             conv2_w[0], conv2_w[1], conv2_w[2], conv2_b.reshape(1, 256))


def _head_proj(a):
    h_heads, hdim = a.shape
    eye = jnp.eye(h_heads, dtype=a.dtype)
    blk = (a[:, :, None] * eye[:, None, :]).reshape(h_heads * hdim, h_heads)
    return jnp.concatenate(
        [blk, jnp.zeros((h_heads * hdim, 16 - h_heads), a.dtype)], axis=1)


def _pad_wsec(wsec):
    sec, h_heads = wsec.shape
    return jnp.concatenate(
        [wsec, jnp.zeros((sec, 16 - h_heads), wsec.dtype)], axis=1)


def kernel(entities, actions, edge_index, security_features, entity_emb,
           action_emb, W1, a_src1, a_dst1, Wsec1, b1, W2, a_src2, a_dst2,
           Wsec2, b2, conv1_w, conv1_b, conv2_w, conv2_b):
    n = entities.shape[0]
    npad = _ceil_to(n, max(_NW * _CHE, _NS * _CH))

    ent_p = jnp.concatenate(
        [entities.astype(jnp.int32), jnp.zeros((npad - n,), jnp.int32)])
    act_p = jnp.concatenate(
        [actions.astype(jnp.int32), jnp.zeros((npad - n,), jnp.int32)])
    sec_p = jnp.concatenate(
        [security_features,
         jnp.zeros((npad - n, security_features.shape[1]), jnp.float32)])
    ei = edge_index.astype(jnp.int32)

    xe, xa = _embed_gather(ent_p, act_p, entity_emb, action_emb, npad)

    zero_b = jnp.zeros((1, 256), jnp.float32)
    h0, h1, est, edt = _proj(xe, xa, sec_p, zero_b, W1, _head_proj(a_src1),
                             _head_proj(a_dst1), _pad_wsec(Wsec1),
                             apply_elu=False)
    alpha1_p, g0, g1 = _edge_phase(est, edt, h0, h1, ei)

    h0b, h1b, est2, edt2 = _proj(g0, g1, sec_p, b1.reshape(1, 256), W2,
                                 _head_proj(a_src2), _head_proj(a_dst2),
                                 _pad_wsec(Wsec2), apply_elu=True)
    alpha2_p, f0, f1 = _edge_phase(est2, edt2, h0b, h1b, ei)

    y = _convs(f0, f1, b2.reshape(1, 256), conv1_w, conv1_b, conv2_w, conv2_b,
               n_out=n)
    e_total = ei.shape[1]
    att1 = alpha1_p.reshape(e_total, 16)[:, :8]
    att2 = alpha2_p.reshape(e_total, 16)[:, :8]
    return y, att1, att2
